# baseline (speedup 1.0000x reference)
import jax
import jax.numpy as jnp
from jax import lax
from jax.experimental import pallas as pl
from jax.experimental.pallas import tpu as pltpu
from jax.experimental.pallas import tpu_sc as plsc

_NODE = {"cats": 100, "subcats": 1000, "depts": 5000, "subdepts": 20000,
         "classes": 50000, "products": 100000, "vendors": 10000}
_EDGES = [("subcats", "cats", 1000), ("depts", "subcats", 5000),
          ("subdepts", "depts", 20000), ("classes", "subdepts", 50000),
          ("products", "classes", 100000), ("vendors", "products", 200000)]
_D = 128
_BLK = 128
_NTILES = 32
_EDGE_Q = _BLK * _NTILES
_HSLAB = 32
_UMAX = 11008
_REG = _UMAX + 128


def _rup(x, m):
    return (x + m - 1) // m * m


_NDST = [_NODE[d] for (_, d, _) in _EDGES]
_NSRC = [_NODE[s] for (s, _, _) in _EDGES]
_EPAD = [_rup(e, 2 * _EDGE_Q) for (_, _, e) in _EDGES]
_BPT = [ep // _EDGE_Q for ep in _EPAD]
_NPAD = [_rup(n + 1, 128) for n in _NDST]
_PASS = [[(lo, min(_UMAX, np_ - lo)) for lo in range(0, np_, _UMAX)]
         for np_ in _NPAD]
_NWORK = sum(len(p) for p in _PASS)


def _sc_body(*refs):
    xs = refs[0:6]
    zeros = refs[6]
    ones = refs[7]
    srcs = refs[8:14]
    dsts = refs[14:14 + _NWORK]
    aggs = refs[14 + _NWORK:20 + _NWORK]
    cnts = refs[20 + _NWORK:26 + _NWORK]
    (S, isrc, idst, rows0, rows1,
     sg0, sg1, ss0, ss1) = refs[26 + _NWORK:35 + _NWORK]

    c = lax.axis_index("c")
    s = lax.axis_index("s")
    wid = s * 2 + c

    w = 0
    for r in range(6):
        bpt = _BPT[r]

        def _scan_edges(dw, do_gather):
            for h0 in range(0, bpt, _HSLAB):
                hs = min(_HSLAB, bpt - h0)
                pltpu.sync_copy(dsts[dw].at[wid, pl.ds(h0, hs)],
                                idst.at[pl.ds(0, hs)])
                if do_gather:
                    pltpu.sync_copy(srcs[r].at[wid, pl.ds(h0, hs)],
                                    isrc.at[pl.ds(0, hs)])

                    @pl.loop(0, hs // 2)
                    def _(t):
                        j0 = 2 * t
                        g0 = pltpu.async_copy(xs[r].at[isrc.at[j0]],
                                              rows0, sg0)
                        g1 = pltpu.async_copy(xs[r].at[isrc.at[j0 + 1]],
                                              rows1, sg1)
                        g0.wait()
                        c0 = pltpu.async_copy(rows0, S.at[idst.at[j0]],
                                              ss0, add=True)
                        g1.wait()
                        c1 = pltpu.async_copy(rows1, S.at[idst.at[j0 + 1]],
                                              ss1, add=True)
                        c0.wait()
                        c1.wait()
                else:
                    @pl.loop(0, hs // 2)
                    def _(t):
                        j0 = 2 * t
                        c0 = pltpu.async_copy(rows0, S.at[idst.at[j0]],
                                              ss0, add=True)
                        c1 = pltpu.async_copy(rows0, S.at[idst.at[j0 + 1]],
                                              ss1, add=True)
                        c0.wait()
                        c1.wait()

        for lo, u in _PASS[r]:
            rz = (u + 128) // 16
            rd = u // 16

            pltpu.sync_copy(zeros.at[pl.ds(0, rz)], S.at[pl.ds(s * rz, rz)])
            pltpu.sync_copy(ones, rows0)
            plsc.subcore_barrier()
            _scan_edges(w, do_gather=False)
            plsc.subcore_barrier()
            pltpu.sync_copy(S.at[pl.ds(s * rd, rd)],
                            cnts[r].at[c, pl.ds(lo + s * rd, rd)])
            plsc.subcore_barrier()

            pltpu.sync_copy(zeros.at[pl.ds(0, rz)], S.at[pl.ds(s * rz, rz)])
            plsc.subcore_barrier()
            _scan_edges(w, do_gather=True)
            plsc.subcore_barrier()
            pltpu.sync_copy(S.at[pl.ds(s * rd, rd)],
                            aggs[r].at[c, pl.ds(lo + s * rd, rd)])
            plsc.subcore_barrier()
            w += 1


def _sc_aggregate(xs, zeros, ones, srcs, dsts):
    out_type = ([jax.ShapeDtypeStruct((2, np_, _D), jnp.float32)
                 for np_ in _NPAD]
                + [jax.ShapeDtypeStruct((2, np_, _D), jnp.float32)
                   for np_ in _NPAD])
    mesh = plsc.VectorSubcoreMesh(core_axis_name="c", subcore_axis_name="s")
    kfn = pl.kernel(
        _sc_body,
        out_type=out_type,
        mesh=mesh,
        scratch_types=[
            pltpu.VMEM_SHARED((_REG, _D), jnp.float32),
            pltpu.VMEM((_HSLAB, _BLK), jnp.int32),
            pltpu.VMEM((_HSLAB, _BLK), jnp.int32),
            pltpu.VMEM((_BLK, _D), jnp.float32),
            pltpu.VMEM((_BLK, _D), jnp.float32),
            pltpu.SemaphoreType.DMA,
            pltpu.SemaphoreType.DMA,
            pltpu.SemaphoreType.DMA,
            pltpu.SemaphoreType.DMA,
        ],
    )
    res = kfn(*xs, zeros, ones, *srcs, *dsts)
    return res[0:6], res[6:12]


def _tc_body(agg_ref, cnt_ref, xd_ref, wl_ref, bl_ref, wr_ref, wlin_ref,
             blin_ref, out_ref):
    p = agg_ref[0] + agg_ref[1]
    cnt = cnt_ref[0, :, 0:1] + cnt_ref[1, :, 0:1]
    mean = p / jnp.maximum(cnt, 1.0)
    hi = lax.Precision.HIGHEST
    h = (lax.dot_general(mean, wl_ref[...], (((1,), (1,)), ((), ())),
                         precision=hi)
         + bl_ref[...]
         + lax.dot_general(xd_ref[...], wr_ref[...], (((1,), (1,)), ((), ())),
                           precision=hi))
    out_ref[...] = (lax.dot_general(h, wlin_ref[...], (((1,), (1,)), ((), ())),
                                    precision=hi)
                    + blin_ref[...])


def _tc_sage(agg, cnt, x_dst, Wl, bl, Wr, W_lin, b_lin, n_dst):
    bs = min(512, _rup(n_dst, 8))
    grid = (pl.cdiv(n_dst, bs),)
    full = lambda i: (0, 0)
    return pl.pallas_call(
        _tc_body,
        grid=grid,
        in_specs=[
            pl.BlockSpec((2, bs, _D), lambda i: (0, i, 0)),
            pl.BlockSpec((2, bs, _D), lambda i: (0, i, 0)),
            pl.BlockSpec((bs, _D), lambda i: (i, 0)),
            pl.BlockSpec((_D, _D), full),
            pl.BlockSpec((1, _D), full),
            pl.BlockSpec((_D, _D), full),
            pl.BlockSpec((_D, _D), full),
            pl.BlockSpec((1, _D), full),
        ],
        out_specs=pl.BlockSpec((bs, _D), lambda i: (i, 0)),
        out_shape=jax.ShapeDtypeStruct((n_dst, _D), jnp.float32),
    )(agg, cnt, x_dst, Wl, bl.reshape(1, _D), Wr, W_lin, b_lin.reshape(1, _D))


@jax.jit
def kernel(x_cats, x_subcats, x_depts, x_subdepts, x_classes, x_products,
           x_vendors, edge_index_0, edge_index_1, edge_index_2, edge_index_3,
           edge_index_4, edge_index_5, Wl_0, bl_0, Wr_0, Wl_1, bl_1, Wr_1,
           Wl_2, bl_2, Wr_2, Wl_3, bl_3, Wr_3, Wl_4, bl_4, Wr_4, Wl_5, bl_5,
           Wr_5, W_lin, b_lin):
    xd = {"cats": x_cats, "subcats": x_subcats, "depts": x_depts,
          "subdepts": x_subdepts, "classes": x_classes,
          "products": x_products, "vendors": x_vendors}
    eis = [edge_index_0, edge_index_1, edge_index_2, edge_index_3,
           edge_index_4, edge_index_5]
    Wls = [Wl_0, Wl_1, Wl_2, Wl_3, Wl_4, Wl_5]
    bls = [bl_0, bl_1, bl_2, bl_3, bl_4, bl_5]
    Wrs = [Wr_0, Wr_1, Wr_2, Wr_3, Wr_4, Wr_5]

    xsl, srcs, dsts = [], [], []
    for r, (sname, dname, e) in enumerate(_EDGES):
        xsl.append(xd[sname])
        pad = _EPAD[r] - e
        src = jnp.pad(eis[r][0], (0, pad))
        dst = jnp.pad(eis[r][1], (0, pad),
                      constant_values=_NDST[r])
        srcs.append(src.reshape(_NTILES, -1, _BLK))
        for lo, u in _PASS[r]:
            inw = (dst >= lo) & (dst < lo + u)
            dw = jnp.where(inw, dst - lo, _UMAX)
            dsts.append(dw.reshape(_NTILES, -1, _BLK))

    zeros = jnp.zeros((_REG // 16, _D), jnp.float32)
    ones = jnp.ones((_BLK, _D), jnp.float32)
    aggs, cnts = _sc_aggregate(xsl, zeros, ones, srcs, dsts)

    outs = []
    for r, (sname, dname, e) in enumerate(_EDGES):
        outs.append(_tc_sage(aggs[r], cnts[r], xd[dname], Wls[r], bls[r],
                             Wrs[r], W_lin, b_lin, _NDST[r]))
    return tuple(outs)


# speedup vs baseline: 1.7855x; 1.7855x over previous
import jax
import jax.numpy as jnp
from jax import lax
from jax.experimental import pallas as pl
from jax.experimental.pallas import tpu as pltpu
from jax.experimental.pallas import tpu_sc as plsc

_NODE = {"cats": 100, "subcats": 1000, "depts": 5000, "subdepts": 20000,
         "classes": 50000, "products": 100000, "vendors": 10000}
_EDGES = [("subcats", "cats", 1000), ("depts", "subcats", 5000),
          ("subdepts", "depts", 20000), ("classes", "subdepts", 50000),
          ("products", "classes", 100000), ("vendors", "products", 200000)]
_D = 128
_BLK = 128
_NTILES = 32
_EDGE_Q = _BLK * _NTILES
_HSLAB = 32
_UMAX = 13056
_REG = _UMAX + 128


def _rup(x, m):
    return (x + m - 1) // m * m


_NDST = [_NODE[d] for (_, d, _) in _EDGES]
_NSRC = [_NODE[s] for (s, _, _) in _EDGES]
_EPAD = [_rup(e, _EDGE_Q) for (_, _, e) in _EDGES]
_BPT = [ep // _EDGE_Q for ep in _EPAD]
_NPAD = [_rup(n + 1, 128) for n in _NDST]
_PASS = [[(lo, min(_UMAX, np_ - lo)) for lo in range(0, np_, _UMAX)]
         for np_ in _NPAD]
_NWORK = sum(len(p) for p in _PASS)


def _sc_body(*refs):
    xs = refs[0:6]
    zeros = refs[6]
    ones = refs[7]
    srcs = refs[8:14]
    dsts = refs[14:14 + _NWORK]
    aggs = refs[14 + _NWORK:20 + _NWORK]
    cnts = refs[20 + _NWORK:26 + _NWORK]
    S, isrc, idst, rows = refs[26 + _NWORK:30 + _NWORK]

    c = lax.axis_index("c")
    s = lax.axis_index("s")
    wid = s * 2 + c

    w = 0
    for r in range(6):
        bpt = _BPT[r]

        def _scan_edges(dw, do_gather):
            for h0 in range(0, bpt, _HSLAB):
                hs = min(_HSLAB, bpt - h0)
                pltpu.sync_copy(dsts[dw].at[wid, pl.ds(h0, hs)],
                                idst.at[pl.ds(0, hs)])
                if do_gather:
                    pltpu.sync_copy(srcs[r].at[wid, pl.ds(h0, hs)],
                                    isrc.at[pl.ds(0, hs)])

                    @pl.loop(0, hs)
                    def _(j):
                        pltpu.sync_copy(xs[r].at[isrc.at[j]], rows)
                        pltpu.sync_copy(rows, S.at[idst.at[j]], add=True)
                else:
                    @pl.loop(0, hs)
                    def _(j):
                        pltpu.sync_copy(rows, S.at[idst.at[j]], add=True)

        for lo, u in _PASS[r]:
            rz = (u + 128) // 16
            rd = u // 16

            pltpu.sync_copy(zeros.at[pl.ds(0, rz)], S.at[pl.ds(s * rz, rz)])
            pltpu.sync_copy(ones, rows)
            plsc.subcore_barrier()
            _scan_edges(w, do_gather=False)
            plsc.subcore_barrier()
            pltpu.sync_copy(S.at[pl.ds(s * rd, rd)],
                            cnts[r].at[c, pl.ds(lo + s * rd, rd)])
            plsc.subcore_barrier()

            pltpu.sync_copy(zeros.at[pl.ds(0, rz)], S.at[pl.ds(s * rz, rz)])
            plsc.subcore_barrier()
            _scan_edges(w, do_gather=True)
            plsc.subcore_barrier()
            pltpu.sync_copy(S.at[pl.ds(s * rd, rd)],
                            aggs[r].at[c, pl.ds(lo + s * rd, rd)])
            plsc.subcore_barrier()
            w += 1


def _sc_aggregate(xs, zeros, ones, srcs, dsts):
    out_type = ([jax.ShapeDtypeStruct((2, np_, _D), jnp.float32)
                 for np_ in _NPAD]
                + [jax.ShapeDtypeStruct((2, np_, _D), jnp.float32)
                   for np_ in _NPAD])
    mesh = plsc.VectorSubcoreMesh(core_axis_name="c", subcore_axis_name="s")
    kfn = pl.kernel(
        _sc_body,
        out_type=out_type,
        mesh=mesh,
        scratch_types=[
            pltpu.VMEM_SHARED((_REG, _D), jnp.float32),
            pltpu.VMEM((_HSLAB, _BLK), jnp.int32),
            pltpu.VMEM((_HSLAB, _BLK), jnp.int32),
            pltpu.VMEM((_BLK, _D), jnp.float32),
        ],
    )
    res = kfn(*xs, zeros, ones, *srcs, *dsts)
    return res[0:6], res[6:12]


def _tc_body(agg_ref, cnt_ref, xd_ref, wl_ref, bl_ref, wr_ref, wlin_ref,
             blin_ref, out_ref):
    p = agg_ref[0] + agg_ref[1]
    cnt = cnt_ref[0, :, 0:1] + cnt_ref[1, :, 0:1]
    mean = p / jnp.maximum(cnt, 1.0)
    hi = lax.Precision.HIGHEST
    h = (lax.dot_general(mean, wl_ref[...], (((1,), (1,)), ((), ())),
                         precision=hi)
         + bl_ref[...]
         + lax.dot_general(xd_ref[...], wr_ref[...], (((1,), (1,)), ((), ())),
                           precision=hi))
    out_ref[...] = (lax.dot_general(h, wlin_ref[...], (((1,), (1,)), ((), ())),
                                    precision=hi)
                    + blin_ref[...])


def _tc_sage(agg, cnt, x_dst, Wl, bl, Wr, W_lin, b_lin, n_dst):
    bs = min(512, _rup(n_dst, 8))
    grid = (pl.cdiv(n_dst, bs),)
    full = lambda i: (0, 0)
    return pl.pallas_call(
        _tc_body,
        grid=grid,
        in_specs=[
            pl.BlockSpec((2, bs, _D), lambda i: (0, i, 0)),
            pl.BlockSpec((2, bs, _D), lambda i: (0, i, 0)),
            pl.BlockSpec((bs, _D), lambda i: (i, 0)),
            pl.BlockSpec((_D, _D), full),
            pl.BlockSpec((1, _D), full),
            pl.BlockSpec((_D, _D), full),
            pl.BlockSpec((_D, _D), full),
            pl.BlockSpec((1, _D), full),
        ],
        out_specs=pl.BlockSpec((bs, _D), lambda i: (i, 0)),
        out_shape=jax.ShapeDtypeStruct((n_dst, _D), jnp.float32),
    )(agg, cnt, x_dst, Wl, bl.reshape(1, _D), Wr, W_lin, b_lin.reshape(1, _D))


@jax.jit
def kernel(x_cats, x_subcats, x_depts, x_subdepts, x_classes, x_products,
           x_vendors, edge_index_0, edge_index_1, edge_index_2, edge_index_3,
           edge_index_4, edge_index_5, Wl_0, bl_0, Wr_0, Wl_1, bl_1, Wr_1,
           Wl_2, bl_2, Wr_2, Wl_3, bl_3, Wr_3, Wl_4, bl_4, Wr_4, Wl_5, bl_5,
           Wr_5, W_lin, b_lin):
    xd = {"cats": x_cats, "subcats": x_subcats, "depts": x_depts,
          "subdepts": x_subdepts, "classes": x_classes,
          "products": x_products, "vendors": x_vendors}
    eis = [edge_index_0, edge_index_1, edge_index_2, edge_index_3,
           edge_index_4, edge_index_5]
    Wls = [Wl_0, Wl_1, Wl_2, Wl_3, Wl_4, Wl_5]
    bls = [bl_0, bl_1, bl_2, bl_3, bl_4, bl_5]
    Wrs = [Wr_0, Wr_1, Wr_2, Wr_3, Wr_4, Wr_5]

    xsl, srcs, dsts = [], [], []
    for r, (sname, dname, e) in enumerate(_EDGES):
        xsl.append(xd[sname])
        pad = _EPAD[r] - e
        src = jnp.pad(eis[r][0], (0, pad))
        dst = jnp.pad(eis[r][1], (0, pad),
                      constant_values=_NDST[r])
        srcs.append(src.reshape(_NTILES, -1, _BLK))
        for lo, u in _PASS[r]:
            inw = (dst >= lo) & (dst < lo + u)
            dw = jnp.where(inw, dst - lo, _UMAX)
            dsts.append(dw.reshape(_NTILES, -1, _BLK))

    zeros = jnp.zeros((_REG // 16, _D), jnp.float32)
    ones = jnp.ones((_BLK, _D), jnp.float32)
    aggs, cnts = _sc_aggregate(xsl, zeros, ones, srcs, dsts)

    outs = []
    for r, (sname, dname, e) in enumerate(_EDGES):
        outs.append(_tc_sage(aggs[r], cnts[r], xd[dname], Wls[r], bls[r],
                             Wrs[r], W_lin, b_lin, _NDST[r]))
    return tuple(outs)
